# Initial kernel scaffold; baseline (speedup 1.0000x reference)
import jax
import jax.numpy as jnp
from jax.experimental import pallas as pl


def kernel(source_pc, target_pc):
    raise NotImplementedError("write your pallas kernel here")


# speedup vs baseline: 8.6577x; 8.6577x over previous
import jax
import jax.numpy as jnp
from jax.experimental import pallas as pl
from jax.experimental.pallas import tpu as pltpu

_K = 3
_OUT_H, _OUT_W = 32, 512
_N = _OUT_H * _OUT_W
_QB = 256
_RB = 2048
_LANES = 128


def _merge3(a, b):
    a1, a2, a3 = a
    b1, b2, b3 = b
    x = jnp.maximum(a1, b1)
    y = jnp.minimum(a2, b2)
    c1 = jnp.minimum(a1, b1)
    c2 = jnp.minimum(x, y)
    c3 = jnp.minimum(jnp.minimum(jnp.maximum(x, y), jnp.maximum(a2, b2)),
                     jnp.minimum(a3, b3))
    return c1, c2, c3


def _block_top3(d):
    w = d.shape[1] // 2
    p1 = jnp.minimum(d[:, :w], d[:, w:])
    p2 = jnp.maximum(d[:, :w], d[:, w:])
    w //= 2
    a1, a2 = p1[:, :w], p2[:, :w]
    b1, b2 = p1[:, w:], p2[:, w:]
    x = jnp.maximum(a1, b1)
    y = jnp.minimum(a2, b2)
    t = (jnp.minimum(a1, b1),
         jnp.minimum(x, y),
         jnp.minimum(jnp.maximum(x, y), jnp.maximum(a2, b2)))
    w //= 2
    while w >= _LANES:
        t = _merge3(tuple(v[:, :w] for v in t), tuple(v[:, w:] for v in t))
        w //= 2
    return t


def _knn_body(q_ref, t_ref, sum_ref, cnt_ref):
    q = q_ref[0]
    q0, q1, q2 = q[:, 0:1], q[:, 1:2], q[:, 2:3]
    qvalid = ((q0 != 0.0) | (q1 != 0.0) | (q2 != 0.0)).astype(jnp.float32)
    qq = q0 * q0 + q1 * q1 + q2 * q2
    q0b = (2.0 * q0).astype(jnp.bfloat16).astype(jnp.float32)
    q1b = (2.0 * q1).astype(jnp.bfloat16).astype(jnp.float32)
    q2b = (2.0 * q2).astype(jnp.bfloat16).astype(jnp.float32)
    inf = jnp.float32(jnp.inf)
    init = (jnp.full((_QB, _LANES), inf, jnp.float32),
            jnp.full((_QB, _LANES), inf, jnp.float32),
            jnp.full((_QB, _LANES), inf, jnp.float32))

    def step(i, m):
        r0 = t_ref[0, 0:1, pl.ds(i * _RB, _RB)]
        r1 = t_ref[0, 1:2, pl.ds(i * _RB, _RB)]
        r2 = t_ref[0, 2:3, pl.ds(i * _RB, _RB)]
        rr = r0 * r0 + r1 * r1 + r2 * r2
        r0b = r0.astype(jnp.bfloat16).astype(jnp.float32)
        r1b = r1.astype(jnp.bfloat16).astype(jnp.float32)
        r2b = r2.astype(jnp.bfloat16).astype(jnp.float32)
        d = (qq + rr) - (q0b * r0b + q1b * r1b + q2b * r2b)
        rvalid = (r0 != 0.0) | (r1 != 0.0) | (r2 != 0.0)
        d = jnp.where(rvalid, jnp.maximum(d, 1e-12), inf)
        return _merge3(m, _block_top3(d))

    m = jax.lax.fori_loop(0, _N // _RB, step, init)
    w = _LANES // 2
    while w >= 1:
        m = _merge3(tuple(v[:, :w] for v in m), tuple(v[:, w:] for v in m))
        w //= 2
    dsum = jnp.sqrt(m[0]) + jnp.sqrt(m[1]) + jnp.sqrt(m[2])
    sum_ref[0, 0, 0] = jnp.sum(dsum * qvalid)
    cnt_ref[0, 0, 0] = jnp.sum(qvalid)


def kernel(source_pc, target_pc):
    B = source_pc.shape[0]
    q = source_pc[:, ::2, ::2, :].reshape(B, _N, 3)
    t = target_pc[:, :, ::2, ::2].reshape(B, 3, _N)
    nq = _N // _QB
    sums, cnts = pl.pallas_call(
        _knn_body,
        grid=(B, nq),
        in_specs=[
            pl.BlockSpec((1, _QB, 3), lambda b, i: (b, i, 0)),
            pl.BlockSpec((1, 3, _N), lambda b, i: (b, 0, 0)),
        ],
        out_specs=[
            pl.BlockSpec((1, 1, 1), lambda b, i: (b * nq + i, 0, 0),
                         memory_space=pltpu.SMEM),
            pl.BlockSpec((1, 1, 1), lambda b, i: (b * nq + i, 0, 0),
                         memory_space=pltpu.SMEM),
        ],
        out_shape=[
            jax.ShapeDtypeStruct((B * nq, 1, 1), jnp.float32),
            jax.ShapeDtypeStruct((B * nq, 1, 1), jnp.float32),
        ],
    )(q, t)
    total = jnp.sum(sums.reshape(B, nq), axis=1)
    count = jnp.sum(cnts.reshape(B, nq), axis=1) * _K
    return jnp.mean(total / count)


# speedup vs baseline: 14.0445x; 1.6222x over previous
import jax
import jax.numpy as jnp
from jax.experimental import pallas as pl
from jax.experimental.pallas import tpu as pltpu

_K = 3
_OUT_H, _OUT_W = 32, 512
_N = _OUT_H * _OUT_W
_QB = 256
_RB = 2048
_LANES = 128


def _merge3(a, b):
    a1, a2, a3 = a
    b1, b2, b3 = b
    x = jnp.maximum(a1, b1)
    y = jnp.minimum(a2, b2)
    c1 = jnp.minimum(a1, b1)
    c2 = jnp.minimum(x, y)
    c3 = jnp.minimum(jnp.minimum(jnp.maximum(x, y), jnp.maximum(a2, b2)),
                     jnp.minimum(a3, b3))
    return c1, c2, c3


def _block_top3(d):
    w = d.shape[1] // 2
    p1 = jnp.minimum(d[:, :w], d[:, w:])
    p2 = jnp.maximum(d[:, :w], d[:, w:])
    w //= 2
    a1, a2 = p1[:, :w], p2[:, :w]
    b1, b2 = p1[:, w:], p2[:, w:]
    x = jnp.maximum(a1, b1)
    y = jnp.minimum(a2, b2)
    t = (jnp.minimum(a1, b1),
         jnp.minimum(x, y),
         jnp.minimum(jnp.maximum(x, y), jnp.maximum(a2, b2)))
    w //= 2
    while w >= _LANES:
        t = _merge3(tuple(v[:, :w] for v in t), tuple(v[:, w:] for v in t))
        w //= 2
    return t


def _knn_body(q_ref, t_ref, sum_ref, cnt_ref):
    q = q_ref[0]
    q0, q1, q2 = q[:, 0:1], q[:, 1:2], q[:, 2:3]
    qvalid = ((q0 != 0.0) | (q1 != 0.0) | (q2 != 0.0)).astype(jnp.float32)
    qq = q0 * q0 + q1 * q1 + q2 * q2
    qb = (2.0 * q).astype(jnp.bfloat16)
    inf = jnp.float32(jnp.inf)
    init = (jnp.full((_QB, _LANES), inf, jnp.float32),
            jnp.full((_QB, _LANES), inf, jnp.float32),
            jnp.full((_QB, _LANES), inf, jnp.float32))

    def step(i, m):
        r = t_ref[0, :, pl.ds(i * _RB, _RB)]
        r0, r1, r2 = r[0:1], r[1:2], r[2:3]
        rr = r0 * r0 + r1 * r1 + r2 * r2
        rvalid = (r0 != 0.0) | (r1 != 0.0) | (r2 != 0.0)
        rrm = jnp.where(rvalid, rr, inf)
        cross = jax.lax.dot_general(
            qb, r.astype(jnp.bfloat16), (((1,), (0,)), ((), ())),
            preferred_element_type=jnp.float32)
        return _merge3(m, _block_top3(rrm - cross))

    m = jax.lax.fori_loop(0, _N // _RB, step, init)
    w = _LANES // 2
    while w >= 1:
        m = _merge3(tuple(v[:, :w] for v in m), tuple(v[:, w:] for v in m))
        w //= 2
    dsum = (jnp.sqrt(jnp.maximum(qq + m[0], 1e-12))
            + jnp.sqrt(jnp.maximum(qq + m[1], 1e-12))
            + jnp.sqrt(jnp.maximum(qq + m[2], 1e-12)))
    sum_ref[0, 0, 0] = jnp.sum(dsum * qvalid)
    cnt_ref[0, 0, 0] = jnp.sum(qvalid)


def kernel(source_pc, target_pc):
    B = source_pc.shape[0]
    q = source_pc[:, ::2, ::2, :].reshape(B, _N, 3)
    t = target_pc[:, :, ::2, ::2].reshape(B, 3, _N)
    nq = _N // _QB
    sums, cnts = pl.pallas_call(
        _knn_body,
        grid=(B, nq),
        in_specs=[
            pl.BlockSpec((1, _QB, 3), lambda b, i: (b, i, 0)),
            pl.BlockSpec((1, 3, _N), lambda b, i: (b, 0, 0)),
        ],
        out_specs=[
            pl.BlockSpec((1, 1, 1), lambda b, i: (b * nq + i, 0, 0),
                         memory_space=pltpu.SMEM),
            pl.BlockSpec((1, 1, 1), lambda b, i: (b * nq + i, 0, 0),
                         memory_space=pltpu.SMEM),
        ],
        out_shape=[
            jax.ShapeDtypeStruct((B * nq, 1, 1), jnp.float32),
            jax.ShapeDtypeStruct((B * nq, 1, 1), jnp.float32),
        ],
    )(q, t)
    total = jnp.sum(sums.reshape(B, nq), axis=1)
    count = jnp.sum(cnts.reshape(B, nq), axis=1) * _K
    return jnp.mean(total / count)


# speedup vs baseline: 17.0554x; 1.2144x over previous
import jax
import jax.numpy as jnp
from jax.experimental import pallas as pl
from jax.experimental.pallas import tpu as pltpu

_K = 3
_OUT_H, _OUT_W = 32, 512
_N = _OUT_H * _OUT_W
_QB = 256
_RB = 4096
_LANES = 128


def _merge3(a, b):
    a1, a2, a3 = a
    b1, b2, b3 = b
    x = jnp.maximum(a1, b1)
    y = jnp.minimum(a2, b2)
    return (jnp.minimum(a1, b1),
            jnp.minimum(x, y),
            jnp.minimum(jnp.maximum(x, y), jnp.minimum(a3, b3)))


def _block_top3(d):
    w = d.shape[1] // 2
    p1 = jnp.minimum(d[:, :w], d[:, w:])
    p2 = jnp.maximum(d[:, :w], d[:, w:])
    w //= 2
    a1, a2 = p1[:, :w], p2[:, :w]
    b1, b2 = p1[:, w:], p2[:, w:]
    x = jnp.maximum(a1, b1)
    y = jnp.minimum(a2, b2)
    t = (jnp.minimum(a1, b1), jnp.minimum(x, y), jnp.maximum(x, y))
    w //= 2
    while w >= _LANES:
        t = _merge3(tuple(v[:, :w] for v in t), tuple(v[:, w:] for v in t))
        w //= 2
    return t


def _knn_body(q_ref, t_ref, sum_ref, cnt_ref):
    q = q_ref[0]
    q0, q1, q2 = q[:, 0:1], q[:, 1:2], q[:, 2:3]
    qvalid = ((q0 != 0.0) | (q1 != 0.0) | (q2 != 0.0)).astype(jnp.float32)
    qq = q0 * q0 + q1 * q1 + q2 * q2
    qb = (2.0 * q).astype(jnp.bfloat16)
    inf = jnp.float32(jnp.inf)

    t_all = t_ref[0]
    r0, r1, r2 = t_all[0:1], t_all[1:2], t_all[2:3]
    rr = r0 * r0 + r1 * r1 + r2 * r2
    rvalid = (r0 != 0.0) | (r1 != 0.0) | (r2 != 0.0)
    rrm = jnp.where(rvalid, rr, inf)
    tb = t_all.astype(jnp.bfloat16)

    m = (jnp.full((_QB, _LANES), inf, jnp.float32),
         jnp.full((_QB, _LANES), inf, jnp.float32),
         jnp.full((_QB, _LANES), inf, jnp.float32))
    for i in range(_N // _RB):
        cross = jax.lax.dot_general(
            qb, tb[:, i * _RB:(i + 1) * _RB], (((1,), (0,)), ((), ())),
            preferred_element_type=jnp.float32)
        s = rrm[:, i * _RB:(i + 1) * _RB] - cross
        m = _merge3(m, _block_top3(s))
    w = _LANES // 2
    while w >= 1:
        m = _merge3(tuple(v[:, :w] for v in m), tuple(v[:, w:] for v in m))
        w //= 2
    dsum = (jnp.sqrt(jnp.maximum(qq + m[0], 1e-12))
            + jnp.sqrt(jnp.maximum(qq + m[1], 1e-12))
            + jnp.sqrt(jnp.maximum(qq + m[2], 1e-12)))
    sum_ref[0, 0, 0] = jnp.sum(dsum * qvalid)
    cnt_ref[0, 0, 0] = jnp.sum(qvalid)


def kernel(source_pc, target_pc):
    B = source_pc.shape[0]
    q = source_pc[:, ::2, ::2, :].reshape(B, _N, 3)
    t = target_pc[:, :, ::2, ::2].reshape(B, 3, _N)
    nq = _N // _QB
    sums, cnts = pl.pallas_call(
        _knn_body,
        grid=(B, nq),
        in_specs=[
            pl.BlockSpec((1, _QB, 3), lambda b, i: (b, i, 0)),
            pl.BlockSpec((1, 3, _N), lambda b, i: (b, 0, 0)),
        ],
        out_specs=[
            pl.BlockSpec((1, 1, 1), lambda b, i: (b * nq + i, 0, 0),
                         memory_space=pltpu.SMEM),
            pl.BlockSpec((1, 1, 1), lambda b, i: (b * nq + i, 0, 0),
                         memory_space=pltpu.SMEM),
        ],
        out_shape=[
            jax.ShapeDtypeStruct((B * nq, 1, 1), jnp.float32),
            jax.ShapeDtypeStruct((B * nq, 1, 1), jnp.float32),
        ],
    )(q, t)
    total = jnp.sum(sums.reshape(B, nq), axis=1)
    count = jnp.sum(cnts.reshape(B, nq), axis=1) * _K
    return jnp.mean(total / count)


# speedup vs baseline: 19.3071x; 1.1320x over previous
import jax
import jax.numpy as jnp
from jax.experimental import pallas as pl
from jax.experimental.pallas import tpu as pltpu

_K = 3
_OUT_H, _OUT_W = 32, 512
_N = _OUT_H * _OUT_W
_QB = 512
_RB = 2048
_LANES = 128


def _merge3(a, b):
    a1, a2, a3 = a
    b1, b2, b3 = b
    x = jnp.maximum(a1, b1)
    y = jnp.minimum(a2, b2)
    return (jnp.minimum(a1, b1),
            jnp.minimum(x, y),
            jnp.minimum(jnp.maximum(x, y), jnp.minimum(a3, b3)))


def _block_top3(d):
    w = d.shape[1] // 2
    p1 = jnp.minimum(d[:, :w], d[:, w:])
    p2 = jnp.maximum(d[:, :w], d[:, w:])
    w //= 2
    a1, a2 = p1[:, :w], p2[:, :w]
    b1, b2 = p1[:, w:], p2[:, w:]
    x = jnp.maximum(a1, b1)
    y = jnp.minimum(a2, b2)
    t = (jnp.minimum(a1, b1), jnp.minimum(x, y), jnp.maximum(x, y))
    w //= 2
    while w >= _LANES:
        t = _merge3(tuple(v[:, :w] for v in t), tuple(v[:, w:] for v in t))
        w //= 2
    return t


def _knn_body(q_ref, t_ref, sum_ref, cnt_ref):
    q = q_ref[0]
    q0, q1, q2 = q[:, 0:1], q[:, 1:2], q[:, 2:3]
    qvalid = ((q0 != 0.0) | (q1 != 0.0) | (q2 != 0.0)).astype(jnp.float32)
    qq = q0 * q0 + q1 * q1 + q2 * q2
    qb = (2.0 * q).astype(jnp.bfloat16)
    inf = jnp.float32(jnp.inf)

    t_all = t_ref[0]
    r0, r1, r2 = t_all[0:1], t_all[1:2], t_all[2:3]
    rr = r0 * r0 + r1 * r1 + r2 * r2
    rvalid = (r0 != 0.0) | (r1 != 0.0) | (r2 != 0.0)
    rrm = jnp.where(rvalid, rr, inf)
    tb = t_all.astype(jnp.bfloat16)

    m = (jnp.full((_QB, _LANES), inf, jnp.float32),
         jnp.full((_QB, _LANES), inf, jnp.float32),
         jnp.full((_QB, _LANES), inf, jnp.float32))
    for i in range(_N // _RB):
        cross = jax.lax.dot_general(
            qb, tb[:, i * _RB:(i + 1) * _RB], (((1,), (0,)), ((), ())),
            preferred_element_type=jnp.float32)
        s = rrm[:, i * _RB:(i + 1) * _RB] - cross
        m = _merge3(m, _block_top3(s))
    w = _LANES // 2
    while w >= 1:
        m = _merge3(tuple(v[:, :w] for v in m), tuple(v[:, w:] for v in m))
        w //= 2
    dsum = (jnp.sqrt(jnp.maximum(qq + m[0], 1e-12))
            + jnp.sqrt(jnp.maximum(qq + m[1], 1e-12))
            + jnp.sqrt(jnp.maximum(qq + m[2], 1e-12)))
    sum_ref[0, 0, 0] = jnp.sum(dsum * qvalid)
    cnt_ref[0, 0, 0] = jnp.sum(qvalid)


def kernel(source_pc, target_pc):
    B = source_pc.shape[0]
    q = source_pc[:, ::2, ::2, :].reshape(B, _N, 3)
    t = target_pc[:, :, ::2, ::2].reshape(B, 3, _N)
    nq = _N // _QB
    sums, cnts = pl.pallas_call(
        _knn_body,
        grid=(B, nq),
        in_specs=[
            pl.BlockSpec((1, _QB, 3), lambda b, i: (b, i, 0)),
            pl.BlockSpec((1, 3, _N), lambda b, i: (b, 0, 0)),
        ],
        out_specs=[
            pl.BlockSpec((1, 1, 1), lambda b, i: (b * nq + i, 0, 0),
                         memory_space=pltpu.SMEM),
            pl.BlockSpec((1, 1, 1), lambda b, i: (b * nq + i, 0, 0),
                         memory_space=pltpu.SMEM),
        ],
        out_shape=[
            jax.ShapeDtypeStruct((B * nq, 1, 1), jnp.float32),
            jax.ShapeDtypeStruct((B * nq, 1, 1), jnp.float32),
        ],
    )(q, t)
    total = jnp.sum(sums.reshape(B, nq), axis=1)
    count = jnp.sum(cnts.reshape(B, nq), axis=1) * _K
    return jnp.mean(total / count)


# speedup vs baseline: 19.6023x; 1.0153x over previous
import jax
import jax.numpy as jnp
from jax.experimental import pallas as pl
from jax.experimental.pallas import tpu as pltpu

_K = 3
_OUT_H, _OUT_W = 32, 512
_N = _OUT_H * _OUT_W
_QB = 1024
_RB = 2048
_LANES = 128


def _merge3(a, b):
    a1, a2, a3 = a
    b1, b2, b3 = b
    x = jnp.maximum(a1, b1)
    y = jnp.minimum(a2, b2)
    return (jnp.minimum(a1, b1),
            jnp.minimum(x, y),
            jnp.minimum(jnp.maximum(x, y), jnp.minimum(a3, b3)))


def _block_top3(d):
    w = d.shape[1] // 2
    p1 = jnp.minimum(d[:, :w], d[:, w:])
    p2 = jnp.maximum(d[:, :w], d[:, w:])
    w //= 2
    a1, a2 = p1[:, :w], p2[:, :w]
    b1, b2 = p1[:, w:], p2[:, w:]
    x = jnp.maximum(a1, b1)
    y = jnp.minimum(a2, b2)
    t = (jnp.minimum(a1, b1), jnp.minimum(x, y), jnp.maximum(x, y))
    w //= 2
    while w >= _LANES:
        t = _merge3(tuple(v[:, :w] for v in t), tuple(v[:, w:] for v in t))
        w //= 2
    return t


def _knn_body(q_ref, t_ref, sum_ref, cnt_ref):
    q = q_ref[0]
    q0, q1, q2 = q[:, 0:1], q[:, 1:2], q[:, 2:3]
    qvalid = ((q0 != 0.0) | (q1 != 0.0) | (q2 != 0.0)).astype(jnp.float32)
    qq = q0 * q0 + q1 * q1 + q2 * q2
    qb = (2.0 * q).astype(jnp.bfloat16)
    inf = jnp.float32(jnp.inf)

    t_all = t_ref[0]
    r0, r1, r2 = t_all[0:1], t_all[1:2], t_all[2:3]
    rr = r0 * r0 + r1 * r1 + r2 * r2
    rvalid = (r0 != 0.0) | (r1 != 0.0) | (r2 != 0.0)
    rrm = jnp.where(rvalid, rr, inf)
    tb = t_all.astype(jnp.bfloat16)

    m = (jnp.full((_QB, _LANES), inf, jnp.float32),
         jnp.full((_QB, _LANES), inf, jnp.float32),
         jnp.full((_QB, _LANES), inf, jnp.float32))
    for i in range(_N // _RB):
        cross = jax.lax.dot_general(
            qb, tb[:, i * _RB:(i + 1) * _RB], (((1,), (0,)), ((), ())),
            preferred_element_type=jnp.float32)
        s = rrm[:, i * _RB:(i + 1) * _RB] - cross
        m = _merge3(m, _block_top3(s))
    w = _LANES // 2
    while w >= 1:
        m = _merge3(tuple(v[:, :w] for v in m), tuple(v[:, w:] for v in m))
        w //= 2
    dsum = (jnp.sqrt(jnp.maximum(qq + m[0], 1e-12))
            + jnp.sqrt(jnp.maximum(qq + m[1], 1e-12))
            + jnp.sqrt(jnp.maximum(qq + m[2], 1e-12)))
    sum_ref[0, 0, 0] = jnp.sum(dsum * qvalid)
    cnt_ref[0, 0, 0] = jnp.sum(qvalid)


def kernel(source_pc, target_pc):
    B = source_pc.shape[0]
    q = source_pc[:, ::2, ::2, :].reshape(B, _N, 3)
    t = target_pc[:, :, ::2, ::2].reshape(B, 3, _N)
    nq = _N // _QB
    sums, cnts = pl.pallas_call(
        _knn_body,
        grid=(B, nq),
        in_specs=[
            pl.BlockSpec((1, _QB, 3), lambda b, i: (b, i, 0)),
            pl.BlockSpec((1, 3, _N), lambda b, i: (b, 0, 0)),
        ],
        out_specs=[
            pl.BlockSpec((1, 1, 1), lambda b, i: (b * nq + i, 0, 0),
                         memory_space=pltpu.SMEM),
            pl.BlockSpec((1, 1, 1), lambda b, i: (b * nq + i, 0, 0),
                         memory_space=pltpu.SMEM),
        ],
        out_shape=[
            jax.ShapeDtypeStruct((B * nq, 1, 1), jnp.float32),
            jax.ShapeDtypeStruct((B * nq, 1, 1), jnp.float32),
        ],
    )(q, t)
    total = jnp.sum(sums.reshape(B, nq), axis=1)
    count = jnp.sum(cnts.reshape(B, nq), axis=1) * _K
    return jnp.mean(total / count)
